# baseline (speedup 1.0000x reference)
import functools

import jax
import jax.numpy as jnp
from jax import lax
from jax.experimental import pallas as pl
from jax.experimental.pallas import tpu as pltpu
from jax.experimental.pallas import tpu_sc as plsc

_BATCH = 16384
_DIM = 64
_PAIR = 2 * _DIM
_NUM_CORES = 2
_NUM_SUBCORES = 16
_NUM_WORKERS = _NUM_CORES * _NUM_SUBCORES
_B_PER_W = _BATCH // _NUM_WORKERS
_CHUNK = 256
_N_CHUNKS = _B_PER_W // _CHUNK

_mesh = plsc.VectorSubcoreMesh(core_axis_name="c", subcore_axis_name="s")


@functools.partial(
    pl.kernel,
    mesh=_mesh,
    compiler_params=pltpu.CompilerParams(needs_layout_passes=False),
    out_type=jax.ShapeDtypeStruct((_BATCH, _DIM), jnp.float32),
    scratch_types=[
        pltpu.VMEM((_B_PER_W,), jnp.int32),
        pltpu.VMEM((_CHUNK,), jnp.int32),
        pltpu.VMEM((_CHUNK, _PAIR), jnp.float32),
        pltpu.VMEM((_B_PER_W, _DIM), jnp.float32),
        pltpu.SemaphoreType.DMA,
    ],
)
def _embed_lookup(labels_hbm, pairs_hbm, out_hbm, idx_v, pidx_v, pairs_v, out_v, sem):
    wid = lax.axis_index("s") * _NUM_CORES + lax.axis_index("c")
    base = wid * _B_PER_W
    pltpu.sync_copy(labels_hbm.at[pl.ds(base, _B_PER_W)], idx_v)
    iota = lax.iota(jnp.int32, 16)

    def round_body(r, carry):
        rbase = r * _CHUNK

        def pidx(g, c):
            lv = idx_v[pl.ds(rbase + g * 16, 16)]
            pidx_v[pl.ds(g * 16, 16)] = lax.shift_right_logical(lv, 1)
            return c

        lax.fori_loop(0, _CHUNK // 16, pidx, 0)
        pltpu.async_copy(pairs_hbm.at[pidx_v], pairs_v, sem).wait()

        def select(g, c):
            row_vec = g * 16 + iota
            lv = idx_v[pl.ds(rbase + g * 16, 16)]
            h = (lv & 1) * _DIM
            for col in range(_DIM):
                x = plsc.load_gather(pairs_v, [row_vec, h + col])
                plsc.store_scatter(
                    out_v, [rbase + row_vec, jnp.full((16,), col, jnp.int32)], x
                )
            return c

        lax.fori_loop(0, _CHUNK // 16, select, 0)
        return carry

    lax.fori_loop(0, _N_CHUNKS, round_body, 0)
    pltpu.sync_copy(out_v, out_hbm.at[pl.ds(base, _B_PER_W)])


def kernel(labels, table):
    pairs = table[:1000000].reshape(500000, _PAIR)
    return _embed_lookup(labels.astype(jnp.int32), pairs)


# speedup vs baseline: 1.6137x; 1.6137x over previous
import jax
import jax.numpy as jnp
from jax import lax
from jax.experimental import pallas as pl
from jax.experimental.pallas import tpu as pltpu

_BATCH = 16384
_DIM = 64
_NSEM = 4


def _body(labels_smem, table_hbm, out_vmem, *sems):
    def fire(ci, c):
        base = ci * _NSEM
        for j in range(_NSEM):
            row = labels_smem[base + j]
            pltpu.make_async_copy(
                table_hbm.at[pl.ds(row, 1), :],
                out_vmem.at[pl.ds(base + j, 1), :],
                sems[j],
            ).start()
        return c

    lax.fori_loop(0, _BATCH // _NSEM, fire, 0, unroll=4)
    for j in range(_NSEM):
        pltpu.make_async_copy(
            table_hbm.at[pl.ds(0, _BATCH // _NSEM), :],
            out_vmem.at[pl.ds(0, _BATCH // _NSEM), :],
            sems[j],
        ).wait()


@jax.jit
def kernel(labels, table):
    grid_spec = pltpu.PrefetchScalarGridSpec(
        num_scalar_prefetch=1,
        grid=(1,),
        in_specs=[pl.BlockSpec(memory_space=pltpu.HBM)],
        out_specs=pl.BlockSpec(memory_space=pltpu.VMEM),
        scratch_shapes=[pltpu.SemaphoreType.DMA] * _NSEM,
    )
    return pl.pallas_call(
        _body,
        grid_spec=grid_spec,
        out_shape=jax.ShapeDtypeStruct((_BATCH, _DIM), jnp.float32),
    )(labels.astype(jnp.int32), table)


# speedup vs baseline: 1.8182x; 1.1267x over previous
import functools

import jax
import jax.numpy as jnp
from jax import lax
from jax.experimental import pallas as pl
from jax.experimental.pallas import tpu as pltpu
from jax.experimental.pallas import tpu_sc as plsc

_BATCH = 16384
_DIM = 64
_NUM_CORES = 2
_NUM_SUBCORES = 16
_NUM_WORKERS = _NUM_CORES * _NUM_SUBCORES
_B_PER_W = _BATCH // _NUM_WORKERS

_mesh = plsc.VectorSubcoreMesh(core_axis_name="c", subcore_axis_name="s")


@functools.partial(
    pl.kernel,
    mesh=_mesh,
    out_type=jax.ShapeDtypeStruct((_BATCH, _DIM), jnp.float32),
    scratch_types=[
        pltpu.VMEM((_B_PER_W,), jnp.int32),
        pltpu.VMEM((_B_PER_W, _DIM), jnp.float32),
    ] + [pltpu.SemaphoreType.DMA] * 8,
)
def _embed_lookup(labels_hbm, table_hbm, out_hbm, lbl_v, rows_v, *sems):
    wid = lax.axis_index("s") * _NUM_CORES + lax.axis_index("c")
    base = wid * _B_PER_W
    pltpu.sync_copy(labels_hbm.at[pl.ds(base, _B_PER_W)], lbl_v)

    def fire(ci, carry):
        cbase = ci * 16
        lv = lbl_v[pl.ds(cbase, 16)]
        for j in range(16):
            row = lv[j]
            pltpu.make_async_copy(
                table_hbm.at[pl.ds(row, 1), :],
                rows_v.at[pl.ds(cbase + j, 1), :],
                sems[j % 8],
            ).start()
        return carry

    lax.fori_loop(0, _B_PER_W // 16, fire, 0)

    def drain(i, carry):
        for j in range(8):
            pltpu.make_async_copy(
                table_hbm.at[pl.ds(0, 1), :],
                rows_v.at[pl.ds(0, 1), :],
                sems[j],
            ).wait()
        return carry

    lax.fori_loop(0, _B_PER_W // 8, drain, 0)
    pltpu.sync_copy(rows_v, out_hbm.at[pl.ds(base, _B_PER_W)])


def kernel(labels, table):
    return _embed_lookup(labels.astype(jnp.int32), table)
